# Initial kernel scaffold; baseline (speedup 1.0000x reference)
import jax
import jax.numpy as jnp
from jax.experimental import pallas as pl
from jax.experimental.pallas import tpu as pltpu

H = 12
HD = 64
BLK = 32
NBLK = 4
QBS = 8
SEQ_TILE = 256


def _qkv_body(hs_ref, wq_ref, wk_ref, wv_ref, bq_ref, bk_ref, bv_ref,
              q_ref, k_ref, v_ref, qb_ref, kb_ref):
    hs = hs_ref[...]
    scale = HD ** -0.5
    q = (jnp.dot(hs, wq_ref[...], preferred_element_type=jnp.float32) + bq_ref[...]) * scale
    k = jnp.dot(hs, wk_ref[...], preferred_element_type=jnp.float32) + bk_ref[...]
    v = jnp.dot(hs, wv_ref[...], preferred_element_type=jnp.float32) + bv_ref[...]
    q_ref[...] = q
    k_ref[...] = k
    v_ref[...] = v
    ts = hs.shape[0]
    nb = ts // BLK
    ri = jax.lax.broadcasted_iota(jnp.int32, (nb, ts), 0)
    ci = jax.lax.broadcasted_iota(jnp.int32, (nb, ts), 1)
    sel = jnp.where((ci >= ri * BLK) & (ci < ri * BLK + BLK), 1.0 / BLK, 0.0)
    qb_ref[...] = jnp.dot(sel, q, preferred_element_type=jnp.float32)
    kb_ref[...] = jnp.dot(sel, k, preferred_element_type=jnp.float32)


def _route_body(qb_ref, kb_ref, top_ref):
    nb = qb_ref.shape[0]
    for h in range(H):
        qh = qb_ref[:, h * HD:(h + 1) * HD]
        kh = kb_ref[:, h * HD:(h + 1) * HD]
        s = jax.lax.dot_general(qh, kh, (((1,), (1,)), ((), ())),
                                preferred_element_type=jnp.float32)
        lane = jax.lax.broadcasted_iota(jnp.int32, s.shape, 1)
        lane_out = jax.lax.broadcasted_iota(jnp.int32, (nb, 128), 1)
        top = jnp.zeros((nb, 128), jnp.int32)
        for m in range(NBLK):
            vmax = jnp.max(s, axis=1, keepdims=True)
            cand = jnp.where(s == vmax, lane, nb)
            idx = jnp.min(cand, axis=1, keepdims=True)
            top = jnp.where(lane_out == m, idx, top)
            s = jnp.where(lane == idx, -1e30, s)
        top_ref[h * nb:(h + 1) * nb, :] = top


def _attn_body(top_ref, q_ref, k_ref, v_ref, o_ref):
    h = pl.program_id(0)
    g = pl.program_id(1)
    nqb = 64
    for j in range(QBS):
        qb = g * QBS + j
        row = h * nqb + qb
        ks, vs = [], []
        for m in range(NBLK):
            idx = top_ref[row, m]
            ks.append(k_ref[pl.ds(idx * BLK, BLK), :])
            vs.append(v_ref[pl.ds(idx * BLK, BLK), :])
        ksel = jnp.concatenate(ks, axis=0)
        vsel = jnp.concatenate(vs, axis=0)
        qj = q_ref[j * BLK:(j + 1) * BLK, :]
        s = jax.lax.dot_general(qj, ksel, (((1,), (1,)), ((), ())),
                                preferred_element_type=jnp.float32)
        mx = jnp.max(s, axis=1, keepdims=True)
        e = jnp.exp(s - mx)
        p = e / jnp.sum(e, axis=1, keepdims=True)
        o_ref[j * BLK:(j + 1) * BLK, :] = jax.lax.dot_general(
            p, vsel, (((1,), (0,)), ((), ())), preferred_element_type=jnp.float32)


def _head_body(ctx_ref, wo_ref, bo_ref, wp_ref, bp_ref, ws_ref, wc_ref, bc_ref,
               out_ref):
    ctx = ctx_ref[...]
    ao = jnp.dot(ctx, wo_ref[...], preferred_element_type=jnp.float32) + bo_ref[...]
    hp = jnp.tanh(jnp.dot(ao, wp_ref[...], preferred_element_type=jnp.float32) + bp_ref[...])
    sc = jax.lax.dot_general(hp, ws_ref[...], (((1,), (1,)), ((), ())),
                             preferred_element_type=jnp.float32)
    mx = jnp.max(sc, axis=0, keepdims=True)
    e = jnp.exp(sc - mx)
    p = e / jnp.sum(e, axis=0, keepdims=True)
    pooled = jax.lax.dot_general(p, ao, (((0,), (0,)), ((), ())),
                                 preferred_element_type=jnp.float32)
    out_ref[...] = jnp.dot(pooled, wc_ref[...], preferred_element_type=jnp.float32) + bc_ref[...]


def kernel(hidden_states, attention_mask, Wq, bq, Wk, bk, Wv, bv, Wo, bo,
           Wp, bp, ws, Wc, bc):
    del attention_mask
    bsz, T, DM = hidden_states.shape
    hs = hidden_states.reshape(T, DM)
    nqb = T // BLK
    nlab = Wc.shape[1]
    bq2 = bq.reshape(1, DM)
    bk2 = bk.reshape(1, DM)
    bv2 = bv.reshape(1, DM)
    bo2 = bo.reshape(1, DM)
    bp2 = bp.reshape(1, DM)
    ws2 = ws.reshape(1, DM)
    bc2 = bc.reshape(1, nlab)

    ntile = T // SEQ_TILE
    nb_tile = SEQ_TILE // BLK
    q, k, v, qbm, kbm = pl.pallas_call(
        _qkv_body,
        grid=(ntile,),
        in_specs=[
            pl.BlockSpec((SEQ_TILE, DM), lambda t: (t, 0)),
            pl.BlockSpec((DM, DM), lambda t: (0, 0)),
            pl.BlockSpec((DM, DM), lambda t: (0, 0)),
            pl.BlockSpec((DM, DM), lambda t: (0, 0)),
            pl.BlockSpec((1, DM), lambda t: (0, 0)),
            pl.BlockSpec((1, DM), lambda t: (0, 0)),
            pl.BlockSpec((1, DM), lambda t: (0, 0)),
        ],
        out_specs=[
            pl.BlockSpec((SEQ_TILE, DM), lambda t: (t, 0)),
            pl.BlockSpec((SEQ_TILE, DM), lambda t: (t, 0)),
            pl.BlockSpec((SEQ_TILE, DM), lambda t: (t, 0)),
            pl.BlockSpec((nb_tile, DM), lambda t: (t, 0)),
            pl.BlockSpec((nb_tile, DM), lambda t: (t, 0)),
        ],
        out_shape=[
            jax.ShapeDtypeStruct((T, DM), jnp.float32),
            jax.ShapeDtypeStruct((T, DM), jnp.float32),
            jax.ShapeDtypeStruct((T, DM), jnp.float32),
            jax.ShapeDtypeStruct((nqb, DM), jnp.float32),
            jax.ShapeDtypeStruct((nqb, DM), jnp.float32),
        ],
    )(hs, Wq, Wk, Wv, bq2, bk2, bv2)

    top_full = pl.pallas_call(
        _route_body,
        out_shape=jax.ShapeDtypeStruct((H * nqb, 128), jnp.int32),
    )(qbm, kbm)
    top4 = top_full[:, :NBLK]

    grid_spec = pltpu.PrefetchScalarGridSpec(
        num_scalar_prefetch=1,
        grid=(H, nqb // QBS),
        in_specs=[
            pl.BlockSpec((QBS * BLK, HD), lambda h, g, top: (g, h)),
            pl.BlockSpec((T, HD), lambda h, g, top: (0, h)),
            pl.BlockSpec((T, HD), lambda h, g, top: (0, h)),
        ],
        out_specs=pl.BlockSpec((QBS * BLK, HD), lambda h, g, top: (g, h)),
    )
    ctx = pl.pallas_call(
        _attn_body,
        grid_spec=grid_spec,
        out_shape=jax.ShapeDtypeStruct((T, DM), jnp.float32),
    )(top4, q, k, v)

    logits = pl.pallas_call(
        _head_body,
        out_shape=jax.ShapeDtypeStruct((1, nlab), jnp.float32),
    )(ctx, Wo, bo2, Wp, bp2, ws2, Wc, bc2)
    return logits


# speedup vs baseline: 318.5221x; 318.5221x over previous
import jax
import jax.numpy as jnp
from jax.experimental import pallas as pl
from jax.experimental.pallas import tpu as pltpu

H = 12
HD = 64
BLK = 32
NBLK = 4
QBS = 8
SEQ_TILE = 256


def _mm(a, b, dims=None):
    if dims is None:
        dims = (((a.ndim - 1,), (0,)), ((), ()))
    return jax.lax.dot_general(a.astype(jnp.bfloat16), b.astype(jnp.bfloat16),
                               dims, preferred_element_type=jnp.float32)


def _qkv_body(hs_ref, wq_ref, wk_ref, wv_ref, bq_ref, bk_ref, bv_ref,
              q_ref, k_ref, v_ref, qb_ref, kb_ref):
    hs = hs_ref[...]
    scale = HD ** -0.5
    q = (_mm(hs, wq_ref[...]) + bq_ref[...]) * scale
    k = _mm(hs, wk_ref[...]) + bk_ref[...]
    v = _mm(hs, wv_ref[...]) + bv_ref[...]
    q_ref[...] = q
    k_ref[...] = k
    v_ref[...] = v
    ts, dm = hs.shape
    nb = ts // BLK
    qb_ref[...] = jnp.mean(q.reshape(nb, BLK, dm), axis=1)
    kb_ref[...] = jnp.mean(k.reshape(nb, BLK, dm), axis=1)


def _route_body(qb_ref, kb_ref, top_ref):
    nb = qb_ref.shape[0]
    for h in range(H):
        qh = qb_ref[:, h * HD:(h + 1) * HD]
        kh = kb_ref[:, h * HD:(h + 1) * HD]
        s = _mm(qh, kh, (((1,), (1,)), ((), ())))
        lane = jax.lax.broadcasted_iota(jnp.int32, s.shape, 1)
        lane_out = jax.lax.broadcasted_iota(jnp.int32, (nb, 128), 1)
        top = jnp.zeros((nb, 128), jnp.int32)
        for m in range(NBLK):
            vmax = jnp.max(s, axis=1, keepdims=True)
            cand = jnp.where(s == vmax, lane, nb)
            idx = jnp.min(cand, axis=1, keepdims=True)
            top = jnp.where(lane_out == m, idx, top)
            s = jnp.where(lane == idx, -1e30, s)
        top_ref[h * nb:(h + 1) * nb, :] = top


def _attn_body(top_ref, q_ref, k_ref, v_ref, o_ref):
    hp = pl.program_id(0)
    g = pl.program_id(1)
    nqb = 64
    for j in range(QBS):
        qb = g * QBS + j
        q_all = q_ref[j * BLK:(j + 1) * BLK, :]
        outs = []
        for hh in range(2):
            row = (hp * 2 + hh) * nqb + qb
            ks, vs = [], []
            for m in range(NBLK):
                idx = top_ref[row, m]
                ks.append(k_ref[pl.ds(idx * BLK, BLK), :][:, hh * HD:(hh + 1) * HD])
                vs.append(v_ref[pl.ds(idx * BLK, BLK), :][:, hh * HD:(hh + 1) * HD])
            ksel = jnp.concatenate(ks, axis=0)
            vsel = jnp.concatenate(vs, axis=0)
            qj = q_all[:, hh * HD:(hh + 1) * HD]
            s = _mm(qj, ksel, (((1,), (1,)), ((), ())))
            mx = jnp.max(s, axis=1, keepdims=True)
            e = jnp.exp(s - mx)
            p = e / jnp.sum(e, axis=1, keepdims=True)
            outs.append(_mm(p, vsel, (((1,), (0,)), ((), ()))))
        o_ref[j * BLK:(j + 1) * BLK, :] = jnp.concatenate(outs, axis=1)


def _head_body(ctx_ref, wo_ref, bo_ref, wp_ref, bp_ref, ws_ref, wc_ref, bc_ref,
               out_ref):
    ctx = ctx_ref[...]
    ao = _mm(ctx, wo_ref[...]) + bo_ref[...]
    hp = jnp.tanh(_mm(ao, wp_ref[...]) + bp_ref[...])
    sc = jax.lax.dot_general(hp, ws_ref[...], (((1,), (1,)), ((), ())),
                             preferred_element_type=jnp.float32)
    mx = jnp.max(sc, axis=0, keepdims=True)
    e = jnp.exp(sc - mx)
    p = e / jnp.sum(e, axis=0, keepdims=True)
    pooled = jax.lax.dot_general(p, ao, (((0,), (0,)), ((), ())),
                                 preferred_element_type=jnp.float32)
    out_ref[...] = jax.lax.dot_general(pooled, wc_ref[...],
                                       (((1,), (0,)), ((), ())),
                                       preferred_element_type=jnp.float32) + bc_ref[...]


def kernel(hidden_states, attention_mask, Wq, bq, Wk, bk, Wv, bv, Wo, bo,
           Wp, bp, ws, Wc, bc):
    del attention_mask
    bsz, T, DM = hidden_states.shape
    hs = hidden_states.reshape(T, DM)
    nqb = T // BLK
    nlab = Wc.shape[1]
    bq2 = bq.reshape(1, DM)
    bk2 = bk.reshape(1, DM)
    bv2 = bv.reshape(1, DM)
    bo2 = bo.reshape(1, DM)
    bp2 = bp.reshape(1, DM)
    ws2 = ws.reshape(1, DM)
    bc2 = bc.reshape(1, nlab)

    ntile = T // SEQ_TILE
    nb_tile = SEQ_TILE // BLK
    q, k, v, qbm, kbm = pl.pallas_call(
        _qkv_body,
        grid=(ntile,),
        in_specs=[
            pl.BlockSpec((SEQ_TILE, DM), lambda t: (t, 0)),
            pl.BlockSpec((DM, DM), lambda t: (0, 0)),
            pl.BlockSpec((DM, DM), lambda t: (0, 0)),
            pl.BlockSpec((DM, DM), lambda t: (0, 0)),
            pl.BlockSpec((1, DM), lambda t: (0, 0)),
            pl.BlockSpec((1, DM), lambda t: (0, 0)),
            pl.BlockSpec((1, DM), lambda t: (0, 0)),
        ],
        out_specs=[
            pl.BlockSpec((SEQ_TILE, DM), lambda t: (t, 0)),
            pl.BlockSpec((SEQ_TILE, DM), lambda t: (t, 0)),
            pl.BlockSpec((SEQ_TILE, DM), lambda t: (t, 0)),
            pl.BlockSpec((nb_tile, DM), lambda t: (t, 0)),
            pl.BlockSpec((nb_tile, DM), lambda t: (t, 0)),
        ],
        out_shape=[
            jax.ShapeDtypeStruct((T, DM), jnp.float32),
            jax.ShapeDtypeStruct((T, DM), jnp.float32),
            jax.ShapeDtypeStruct((T, DM), jnp.float32),
            jax.ShapeDtypeStruct((nqb, DM), jnp.float32),
            jax.ShapeDtypeStruct((nqb, DM), jnp.float32),
        ],
    )(hs, Wq, Wk, Wv, bq2, bk2, bv2)

    top_full = pl.pallas_call(
        _route_body,
        out_shape=jax.ShapeDtypeStruct((H * nqb, 128), jnp.int32),
    )(qbm, kbm)
    top4 = top_full[:, :NBLK]

    grid_spec = pltpu.PrefetchScalarGridSpec(
        num_scalar_prefetch=1,
        grid=(H // 2, nqb // QBS),
        in_specs=[
            pl.BlockSpec((QBS * BLK, 2 * HD), lambda hp, g, top: (g, hp)),
            pl.BlockSpec((T, 2 * HD), lambda hp, g, top: (0, hp)),
            pl.BlockSpec((T, 2 * HD), lambda hp, g, top: (0, hp)),
        ],
        out_specs=pl.BlockSpec((QBS * BLK, 2 * HD), lambda hp, g, top: (g, hp)),
    )
    ctx = pl.pallas_call(
        _attn_body,
        grid_spec=grid_spec,
        out_shape=jax.ShapeDtypeStruct((T, DM), jnp.float32),
    )(top4, q, k, v)

    logits = pl.pallas_call(
        _head_body,
        out_shape=jax.ShapeDtypeStruct((1, nlab), jnp.float32),
    )(ctx, Wo, bo2, Wp, bp2, ws2, Wc, bc2)
    return logits
